# baseline (speedup 1.0000x reference)
import functools

import jax
import jax.numpy as jnp
from jax import lax
from jax.experimental import pallas as pl
from jax.experimental.pallas import tpu as pltpu
from jax.experimental.pallas import tpu_sc as plsc

N = 10000
E = 320000
NF = 128
NCLASS = 40

NC = 2
NS = 16
NW = NC * NS
ROWS_PER_TILE = 632
N_PAD = NS * ROWS_PER_TILE
E_PER_TILE = E // NW
CHUNK = 40
N_CHUNKS = E_PER_TILE // CHUNK


CHUNK128 = 128
MAIN_CHUNKS = 78
E_MAIN = NW * MAIN_CHUNKS * CHUNK128
N_CHUNKS_T = MAIN_CHUNKS + 1
E_PAD = NW * N_CHUNKS_T * CHUNK128


def _segsum_body(x_hbm, e_hbm, zero_hbm, out_hbm,
                 acc, srcb, dstb, rows_v, gsem, ssem, isem_s, isem_d):
    src_hbm = e_hbm.at[1]
    dst_hbm = e_hbm.at[0]
    c = lax.axis_index("c")
    s = lax.axis_index("s")
    w = c * NS + s
    r0 = s * ROWS_PER_TILE

    def off(k):
        return jnp.where(k < MAIN_CHUNKS,
                         w * (MAIN_CHUNKS * CHUNK128) + k * CHUNK128,
                         E_MAIN + w * CHUNK128)

    def sh(k):
        return src_hbm.at[pl.ds(off(k), CHUNK128)]

    def dh(k):
        return dst_hbm.at[pl.ds(off(k), CHUNK128)]

    def sv(m):
        return srcb.at[pl.ds(m * CHUNK128, CHUNK128)]

    pltpu.async_copy(sh(0), sv(0), isem_s.at[0])
    pltpu.async_copy(sh(1), sv(1), isem_s.at[1])
    pltpu.async_copy(dh(0), dstb.at[0], isem_d.at[0])
    pltpu.sync_copy(zero_hbm.at[pl.ds(r0, ROWS_PER_TILE)],
                    acc.at[pl.ds(r0, ROWS_PER_TILE)])
    pltpu.make_async_copy(sh(0), sv(0), isem_s.at[0]).wait()
    pltpu.async_copy(x_hbm.at[sv(0)], rows_v.at[0], gsem.at[0])
    plsc.subcore_barrier()

    def step(k, m):
        n = 1 - m
        pltpu.make_async_copy(x_hbm.at[sv(m)], rows_v.at[m], gsem.at[m]).wait()
        pltpu.make_async_copy(dh(k), dstb.at[m], isem_d.at[m]).wait()
        pltpu.async_copy(rows_v.at[m], acc.at[dstb.at[m]],
                         ssem.at[m], add=True)

        @pl.when(k + 2 < N_CHUNKS_T)
        def _():
            pltpu.async_copy(sh(k + 2), sv(m), isem_s.at[m])

        @pl.when(k >= 1)
        def _():
            pltpu.make_async_copy(rows_v.at[n], acc.at[dstb.at[n]],
                                  ssem.at[n]).wait()

        @pl.when(k + 1 < N_CHUNKS_T)
        def _():
            pltpu.async_copy(dh(k + 1), dstb.at[n], isem_d.at[n])
            pltpu.make_async_copy(sh(k + 1), sv(n), isem_s.at[n]).wait()
            pltpu.async_copy(x_hbm.at[sv(n)], rows_v.at[n], gsem.at[n])

    def body(t, carry):
        step(2 * t, 0)
        step(2 * t + 1, 1)
        return carry

    lax.fori_loop(0, MAIN_CHUNKS // 2, body, 0)
    step(MAIN_CHUNKS, MAIN_CHUNKS % 2)
    pltpu.make_async_copy(rows_v.at[0], acc.at[dstb.at[0]],
                          ssem.at[0]).wait()
    plsc.subcore_barrier()
    pltpu.sync_copy(acc.at[pl.ds(r0, ROWS_PER_TILE)],
                    out_hbm.at[c, pl.ds(r0, ROWS_PER_TILE)])


_segsum_call = pl.kernel(
    _segsum_body,
    out_type=jax.ShapeDtypeStruct((NC, N_PAD, NF), jnp.float32),
    mesh=plsc.VectorSubcoreMesh(core_axis_name="c", subcore_axis_name="s"),
    scratch_types=[
        pltpu.VMEM_SHARED((N_PAD, NF), jnp.float32),
        pltpu.VMEM((2 * CHUNK128,), jnp.int32),
        pltpu.VMEM((2, CHUNK128), jnp.int32),
        pltpu.VMEM((2, CHUNK128, NF), jnp.float32),
        pltpu.SemaphoreType.DMA((2,)),
        pltpu.SemaphoreType.DMA((2,)),
        pltpu.SemaphoreType.DMA((2,)),
        pltpu.SemaphoreType.DMA((2,)),
    ],
)


def _elu(a):
    return jnp.where(a > 0, a, jnp.exp(a) - 1.0)


def _mlp_body(p_ref, w1_ref, w2_ref, out_ref):
    a = p_ref[0] + p_ref[1]
    h1 = _elu(jnp.dot(a, w1_ref[...], preferred_element_type=jnp.float32))
    out_ref[...] = jnp.dot(h1, w2_ref[...], preferred_element_type=jnp.float32)


def _head_body(p_ref, fw_ref, fb_ref, out_ref):
    z = _elu(p_ref[0] + p_ref[1])
    logits = jnp.dot(z, fw_ref[...], preferred_element_type=jnp.float32)
    logits = logits + fb_ref[...]
    m = jnp.max(logits, axis=1, keepdims=True)
    lse = jnp.log(jnp.sum(jnp.exp(logits - m), axis=1, keepdims=True)) + m
    out_ref[...] = logits - lse


_BLK = 632
_GRID = N_PAD // _BLK


def _mlp(p, W1, W2):
    return pl.pallas_call(
        _mlp_body,
        grid=(_GRID,),
        in_specs=[
            pl.BlockSpec((2, _BLK, NF), lambda i: (0, i, 0)),
            pl.BlockSpec((NF, NF), lambda i: (0, 0)),
            pl.BlockSpec((NF, NF), lambda i: (0, 0)),
        ],
        out_specs=pl.BlockSpec((_BLK, NF), lambda i: (i, 0)),
        out_shape=jax.ShapeDtypeStruct((N_PAD, NF), jnp.float32),
    )(p, W1, W2)


def _head(p, fc_w, fc_b):
    return pl.pallas_call(
        _head_body,
        grid=(_GRID,),
        in_specs=[
            pl.BlockSpec((2, _BLK, NF), lambda i: (0, i, 0)),
            pl.BlockSpec((NF, NCLASS), lambda i: (0, 0)),
            pl.BlockSpec((1, NCLASS), lambda i: (0, 0)),
        ],
        out_specs=pl.BlockSpec((_BLK, NCLASS), lambda i: (i, 0)),
        out_shape=jax.ShapeDtypeStruct((N, NCLASS), jnp.float32),
    )(p, fc_w, fc_b)


def kernel(x, edge_index, W1, W2, fc_w, fc_b):
    ei = edge_index.astype(jnp.int32)
    pad = jnp.stack([jnp.full((E_PAD - E,), N_PAD - 1, jnp.int32),
                     jnp.zeros((E_PAD - E,), jnp.int32)])
    e = jnp.concatenate([ei, pad], axis=1)
    zeros = jnp.zeros((N_PAD, NF), jnp.float32)

    p = _segsum_call(x, e, zeros)
    h2 = _mlp(p, W1, W2)
    q = _segsum_call(h2, e, zeros)
    return _head(q, fc_w, fc_b.reshape(1, NCLASS))


# speedup vs baseline: 2.0099x; 2.0099x over previous
import functools

import jax
import jax.numpy as jnp
from jax import lax
from jax.experimental import pallas as pl
from jax.experimental.pallas import tpu as pltpu
from jax.experimental.pallas import tpu_sc as plsc

N = 10000
E = 320000
NF = 128
NCLASS = 40

NC = 2
NS = 16
NW = NC * NS
ROWS_PER_TILE = 632
N_PAD = NS * ROWS_PER_TILE
E_PER_TILE = E // NW
CHUNK = 40
N_CHUNKS = E_PER_TILE // CHUNK


NBUF = 5
NGROUPS = N_CHUNKS // NBUF


DEFER = 3
SPLIT = 24


def _segsum_body(x_hbm, e_hbm, zero_hbm, out_hbm,
                 acc, srcb, dstb, rows_v, gsem, gsem2, ssem, isem):
    src_hbm = e_hbm.at[1]
    dst_hbm = e_hbm.at[0]
    c = lax.axis_index("c")
    s = lax.axis_index("s")
    w = c * NS + s
    r0 = s * ROWS_PER_TILE
    pltpu.sync_copy(src_hbm.at[w, 0], srcb.at[0])
    pltpu.sync_copy(dst_hbm.at[w, 0], dstb.at[0])
    pltpu.sync_copy(src_hbm.at[w, 1], srcb.at[1])
    pltpu.sync_copy(dst_hbm.at[w, 1], dstb.at[1])
    pltpu.sync_copy(zero_hbm.at[pl.ds(r0, ROWS_PER_TILE)],
                    acc.at[pl.ds(r0, ROWS_PER_TILE)])
    plsc.subcore_barrier()

    for b in range(NBUF):
        pltpu.async_copy(x_hbm.at[srcb.at[0, b, pl.ds(0, SPLIT)]],
                         rows_v.at[b, pl.ds(0, SPLIT)], gsem.at[b])
        pltpu.async_copy(x_hbm.at[srcb.at[0, b, pl.ds(SPLIT, CHUNK - SPLIT)]],
                         rows_v.at[b, pl.ds(SPLIT, CHUNK - SPLIT)], gsem2.at[b])

    def group(g, p):
        q = (p + 1) % 2

        @pl.when(g >= 2)
        def _():
            pltpu.make_async_copy(
                dst_hbm.at[w, g], dstb.at[p], isem.at[p, 1]).wait()

        for b in range(NBUF):
            i = g * NBUF + b
            pltpu.make_async_copy(
                x_hbm.at[srcb.at[p, b, pl.ds(0, SPLIT)]],
                rows_v.at[b, pl.ds(0, SPLIT)], gsem.at[b]).wait()
            pltpu.make_async_copy(
                x_hbm.at[srcb.at[p, b, pl.ds(SPLIT, CHUNK - SPLIT)]],
                rows_v.at[b, pl.ds(SPLIT, CHUNK - SPLIT)], gsem2.at[b]).wait()
            pltpu.async_copy(rows_v.at[b], acc.at[dstb.at[p, b]],
                             ssem.at[b], add=True)

            if b == 2:
                @pl.when(jnp.logical_and(g >= 1, g + 1 < NGROUPS))
                def _():
                    pltpu.make_async_copy(
                        src_hbm.at[w, g + 1], srcb.at[q], isem.at[q, 0]).wait()

            bj = (b + DEFER) % NBUF
            j = i + DEFER
            pj, pw = (p, q) if b < NBUF - DEFER else (q, p)

            @pl.when(jnp.logical_and(j >= NBUF, j < N_CHUNKS))
            def _():
                pltpu.make_async_copy(
                    rows_v.at[bj], acc.at[dstb.at[pw, bj]], ssem.at[bj]).wait()
                pltpu.async_copy(
                    x_hbm.at[srcb.at[pj, bj, pl.ds(0, SPLIT)]],
                    rows_v.at[bj, pl.ds(0, SPLIT)], gsem.at[bj])
                pltpu.async_copy(
                    x_hbm.at[srcb.at[pj, bj, pl.ds(SPLIT, CHUNK - SPLIT)]],
                    rows_v.at[bj, pl.ds(SPLIT, CHUNK - SPLIT)], gsem2.at[bj])

            if b == 1:
                @pl.when(jnp.logical_and(g + 1 >= 2, g + 1 < NGROUPS))
                def _():
                    pltpu.async_copy(
                        dst_hbm.at[w, g + 1], dstb.at[q], isem.at[q, 1])
            if b == 4:
                @pl.when(jnp.logical_and(g + 2 >= 2, g + 2 < NGROUPS))
                def _():
                    pltpu.async_copy(
                        src_hbm.at[w, g + 2], srcb.at[p], isem.at[p, 0])

    def body(t, carry):
        group(2 * t, 0)
        group(2 * t + 1, 1)
        return carry

    lax.fori_loop(0, NGROUPS // 2, body, 0)
    for b in range(NBUF):
        pltpu.make_async_copy(
            rows_v.at[b], acc.at[dstb.at[(NGROUPS - 1) % 2, b]],
            ssem.at[b]).wait()
    plsc.subcore_barrier()
    pltpu.sync_copy(acc.at[pl.ds(r0, ROWS_PER_TILE)],
                    out_hbm.at[c, pl.ds(r0, ROWS_PER_TILE)])


_segsum_call = pl.kernel(
    _segsum_body,
    out_type=jax.ShapeDtypeStruct((NC, N_PAD, NF), jnp.float32),
    mesh=plsc.VectorSubcoreMesh(core_axis_name="c", subcore_axis_name="s"),
    scratch_types=[
        pltpu.VMEM_SHARED((N_PAD, NF), jnp.float32),
        pltpu.VMEM((2, NBUF, CHUNK), jnp.int32),
        pltpu.VMEM((2, NBUF, CHUNK), jnp.int32),
        pltpu.VMEM((NBUF, CHUNK, NF), jnp.float32),
        pltpu.SemaphoreType.DMA((NBUF,)),
        pltpu.SemaphoreType.DMA((NBUF,)),
        pltpu.SemaphoreType.DMA((NBUF,)),
        pltpu.SemaphoreType.DMA((2, 2)),
    ],
)


def _elu(a):
    return jnp.where(a > 0, a, jnp.exp(a) - 1.0)


def _mlp_body(p_ref, w1_ref, w2_ref, out_ref):
    a = p_ref[0] + p_ref[1]
    h1 = _elu(jnp.dot(a, w1_ref[...], preferred_element_type=jnp.float32))
    out_ref[...] = jnp.dot(h1, w2_ref[...], preferred_element_type=jnp.float32)


def _head_body(p_ref, fw_ref, fb_ref, out_ref):
    z = _elu(p_ref[0] + p_ref[1])
    logits = jnp.dot(z, fw_ref[...], preferred_element_type=jnp.float32)
    logits = logits + fb_ref[...]
    m = jnp.max(logits, axis=1, keepdims=True)
    lse = jnp.log(jnp.sum(jnp.exp(logits - m), axis=1, keepdims=True)) + m
    out_ref[...] = logits - lse


_BLK = 632
_GRID = N_PAD // _BLK


def _mlp(p, W1, W2):
    return pl.pallas_call(
        _mlp_body,
        grid=(_GRID,),
        in_specs=[
            pl.BlockSpec((2, _BLK, NF), lambda i: (0, i, 0)),
            pl.BlockSpec((NF, NF), lambda i: (0, 0)),
            pl.BlockSpec((NF, NF), lambda i: (0, 0)),
        ],
        out_specs=pl.BlockSpec((_BLK, NF), lambda i: (i, 0)),
        out_shape=jax.ShapeDtypeStruct((N_PAD, NF), jnp.float32),
    )(p, W1, W2)


def _head(p, fc_w, fc_b):
    return pl.pallas_call(
        _head_body,
        grid=(_GRID,),
        in_specs=[
            pl.BlockSpec((2, _BLK, NF), lambda i: (0, i, 0)),
            pl.BlockSpec((NF, NCLASS), lambda i: (0, 0)),
            pl.BlockSpec((1, NCLASS), lambda i: (0, 0)),
        ],
        out_specs=pl.BlockSpec((_BLK, NCLASS), lambda i: (i, 0)),
        out_shape=jax.ShapeDtypeStruct((N, NCLASS), jnp.float32),
    )(p, fc_w, fc_b)


def kernel(x, edge_index, W1, W2, fc_w, fc_b):
    e = edge_index.astype(jnp.int32).reshape(2, NW, NGROUPS, NBUF, CHUNK)
    zeros = jnp.zeros((N_PAD, NF), jnp.float32)

    p = _segsum_call(x, e, zeros)
    h2 = _mlp(p, W1, W2)
    q = _segsum_call(h2, e, zeros)
    return _head(q, fc_w, fc_b.reshape(1, NCLASS))


# speedup vs baseline: 2.0757x; 1.0328x over previous
import functools

import jax
import jax.numpy as jnp
from jax import lax
from jax.experimental import pallas as pl
from jax.experimental.pallas import tpu as pltpu
from jax.experimental.pallas import tpu_sc as plsc

N = 10000
E = 320000
NF = 128
NCLASS = 40

NC = 2
NS = 16
NW = NC * NS
ROWS_PER_TILE = 632
N_PAD = NS * ROWS_PER_TILE
E_PER_TILE = E // NW
CHUNK = 40
N_CHUNKS = E_PER_TILE // CHUNK


NBUF = 5
NGROUPS = N_CHUNKS // NBUF


DEFER = 3
SPLIT = 24


def _segsum_body(x_hbm, e_hbm, zero_hbm, out_hbm,
                 acc, srcb, dstb, rows_v, gsem, gsem2, ssem, isem):
    src_hbm = e_hbm.at[1]
    dst_hbm = e_hbm.at[0]
    c = lax.axis_index("c")
    s = lax.axis_index("s")
    w = c * NS + s
    r0 = s * ROWS_PER_TILE
    pltpu.sync_copy(src_hbm.at[w, 0], srcb.at[0])
    pltpu.sync_copy(dst_hbm.at[w, 0], dstb.at[0])
    pltpu.sync_copy(src_hbm.at[w, 1], srcb.at[1])
    pltpu.sync_copy(dst_hbm.at[w, 1], dstb.at[1])
    pltpu.sync_copy(zero_hbm.at[pl.ds(r0, ROWS_PER_TILE)],
                    acc.at[pl.ds(r0, ROWS_PER_TILE)])
    plsc.subcore_barrier()

    for b in range(NBUF):
        pltpu.async_copy(x_hbm.at[srcb.at[0, b, pl.ds(0, SPLIT)]],
                         rows_v.at[b, pl.ds(0, SPLIT)], gsem.at[b])
        pltpu.async_copy(x_hbm.at[srcb.at[0, b, pl.ds(SPLIT, CHUNK - SPLIT)]],
                         rows_v.at[b, pl.ds(SPLIT, CHUNK - SPLIT)], gsem2.at[b])

    def group(g, p):
        q = (p + 1) % 2

        @pl.when(g >= 2)
        def _():
            pltpu.make_async_copy(
                dst_hbm.at[w, g], dstb.at[p], isem.at[p, 1]).wait()

        for b in range(NBUF):
            i = g * NBUF + b
            pltpu.make_async_copy(
                x_hbm.at[srcb.at[p, b, pl.ds(0, SPLIT)]],
                rows_v.at[b, pl.ds(0, SPLIT)], gsem.at[b]).wait()
            pltpu.make_async_copy(
                x_hbm.at[srcb.at[p, b, pl.ds(SPLIT, CHUNK - SPLIT)]],
                rows_v.at[b, pl.ds(SPLIT, CHUNK - SPLIT)], gsem2.at[b]).wait()
            pltpu.async_copy(rows_v.at[b], acc.at[dstb.at[p, b]],
                             ssem.at[b], add=True)

            if b == 2:
                @pl.when(jnp.logical_and(g >= 1, g + 1 < NGROUPS))
                def _():
                    pltpu.make_async_copy(
                        src_hbm.at[w, g + 1], srcb.at[q], isem.at[q, 0]).wait()

            bj = (b + DEFER) % NBUF
            j = i + DEFER
            pj, pw = (p, q) if b < NBUF - DEFER else (q, p)

            @pl.when(jnp.logical_and(j >= NBUF, j < N_CHUNKS))
            def _():
                pltpu.make_async_copy(
                    rows_v.at[bj], acc.at[dstb.at[pw, bj]], ssem.at[bj]).wait()
                pltpu.async_copy(
                    x_hbm.at[srcb.at[pj, bj, pl.ds(0, SPLIT)]],
                    rows_v.at[bj, pl.ds(0, SPLIT)], gsem.at[bj])
                pltpu.async_copy(
                    x_hbm.at[srcb.at[pj, bj, pl.ds(SPLIT, CHUNK - SPLIT)]],
                    rows_v.at[bj, pl.ds(SPLIT, CHUNK - SPLIT)], gsem2.at[bj])

            if b == 1:
                @pl.when(jnp.logical_and(g + 1 >= 2, g + 1 < NGROUPS))
                def _():
                    pltpu.async_copy(
                        dst_hbm.at[w, g + 1], dstb.at[q], isem.at[q, 1])
            if b == 4:
                @pl.when(jnp.logical_and(g + 2 >= 2, g + 2 < NGROUPS))
                def _():
                    pltpu.async_copy(
                        src_hbm.at[w, g + 2], srcb.at[p], isem.at[p, 0])

    def body(t, carry):
        group(2 * t, 0)
        group(2 * t + 1, 1)
        return carry

    lax.fori_loop(0, NGROUPS // 2, body, 0)
    for b in range(NBUF):
        pltpu.make_async_copy(
            rows_v.at[b], acc.at[dstb.at[(NGROUPS - 1) % 2, b]],
            ssem.at[b]).wait()
    plsc.subcore_barrier()
    pltpu.sync_copy(acc.at[pl.ds(r0, ROWS_PER_TILE)],
                    out_hbm.at[c, pl.ds(r0, ROWS_PER_TILE)])


_segsum_call = pl.kernel(
    _segsum_body,
    out_type=jax.ShapeDtypeStruct((NC, N_PAD, NF), jnp.float32),
    mesh=plsc.VectorSubcoreMesh(core_axis_name="c", subcore_axis_name="s"),
    scratch_types=[
        pltpu.VMEM_SHARED((N_PAD, NF), jnp.float32),
        pltpu.VMEM((2, NBUF, CHUNK), jnp.int32),
        pltpu.VMEM((2, NBUF, CHUNK), jnp.int32),
        pltpu.VMEM((NBUF, CHUNK, NF), jnp.float32),
        pltpu.SemaphoreType.DMA((NBUF,)),
        pltpu.SemaphoreType.DMA((NBUF,)),
        pltpu.SemaphoreType.DMA((NBUF,)),
        pltpu.SemaphoreType.DMA((2, 2)),
    ],
)


def _elu(a):
    return jnp.where(a > 0, a, jnp.exp(a) - 1.0)


def _mlp_body(p_ref, w1_ref, w2_ref, out_ref):
    a = p_ref[0] + p_ref[1]
    h1 = _elu(jnp.dot(a, w1_ref[...], preferred_element_type=jnp.float32))
    out_ref[...] = jnp.dot(h1, w2_ref[...], preferred_element_type=jnp.float32)


def _head_body(p_ref, fw_ref, fb_ref, out_ref):
    z = _elu(p_ref[0] + p_ref[1])
    logits = jnp.dot(z, fw_ref[...], preferred_element_type=jnp.float32)
    logits = logits + fb_ref[...]
    m = jnp.max(logits, axis=1, keepdims=True)
    lse = jnp.log(jnp.sum(jnp.exp(logits - m), axis=1, keepdims=True)) + m
    out_ref[...] = logits - lse


_BLK = 1264
_GRID = N_PAD // _BLK


def _mlp(p, W1, W2):
    return pl.pallas_call(
        _mlp_body,
        grid=(_GRID,),
        in_specs=[
            pl.BlockSpec((2, _BLK, NF), lambda i: (0, i, 0)),
            pl.BlockSpec((NF, NF), lambda i: (0, 0)),
            pl.BlockSpec((NF, NF), lambda i: (0, 0)),
        ],
        out_specs=pl.BlockSpec((_BLK, NF), lambda i: (i, 0)),
        out_shape=jax.ShapeDtypeStruct((N_PAD, NF), jnp.float32),
    )(p, W1, W2)


def _head(p, fc_w, fc_b):
    return pl.pallas_call(
        _head_body,
        grid=(_GRID,),
        in_specs=[
            pl.BlockSpec((2, _BLK, NF), lambda i: (0, i, 0)),
            pl.BlockSpec((NF, NCLASS), lambda i: (0, 0)),
            pl.BlockSpec((1, NCLASS), lambda i: (0, 0)),
        ],
        out_specs=pl.BlockSpec((_BLK, NCLASS), lambda i: (i, 0)),
        out_shape=jax.ShapeDtypeStruct((N, NCLASS), jnp.float32),
    )(p, fc_w, fc_b)


def kernel(x, edge_index, W1, W2, fc_w, fc_b):
    e = edge_index.astype(jnp.int32).reshape(2, NW, NGROUPS, NBUF, CHUNK)
    zeros = jnp.zeros((N_PAD, NF), jnp.float32)

    p = _segsum_call(x, e, zeros)
    h2 = _mlp(p, W1, W2)
    q = _segsum_call(h2, e, zeros)
    return _head(q, fc_w, fc_b.reshape(1, NCLASS))


# speedup vs baseline: 2.1284x; 1.0254x over previous
import functools

import jax
import jax.numpy as jnp
from jax import lax
from jax.experimental import pallas as pl
from jax.experimental.pallas import tpu as pltpu
from jax.experimental.pallas import tpu_sc as plsc

N = 10000
E = 320000
NF = 128
NCLASS = 40

NC = 2
NS = 16
NW = NC * NS
ROWS_PER_TILE = 632
N_PAD = NS * ROWS_PER_TILE
E_PER_TILE = E // NW
CHUNK = 40
N_CHUNKS = E_PER_TILE // CHUNK


NBUF = 5
NGROUPS = N_CHUNKS // NBUF


DEFER = 3
SPLIT = 24


def _segsum_body(x_hbm, e_hbm, zero_hbm, out_hbm,
                 acc, srcb, dstb, rows_v, gsem, gsem2, ssem, isem):
    src_hbm = e_hbm.at[1]
    dst_hbm = e_hbm.at[0]
    c = lax.axis_index("c")
    s = lax.axis_index("s")
    w = c * NS + s
    r0 = s * ROWS_PER_TILE
    pltpu.sync_copy(src_hbm.at[w, 0], srcb.at[0])
    pltpu.sync_copy(dst_hbm.at[w, 0], dstb.at[0])
    pltpu.sync_copy(src_hbm.at[w, 1], srcb.at[1])
    pltpu.sync_copy(dst_hbm.at[w, 1], dstb.at[1])
    pltpu.sync_copy(zero_hbm.at[pl.ds(r0, ROWS_PER_TILE)],
                    acc.at[pl.ds(r0, ROWS_PER_TILE)])
    plsc.subcore_barrier()

    for b in range(NBUF):
        pltpu.async_copy(x_hbm.at[srcb.at[0, b, pl.ds(0, SPLIT)]],
                         rows_v.at[b, pl.ds(0, SPLIT)], gsem.at[b])
        pltpu.async_copy(x_hbm.at[srcb.at[0, b, pl.ds(SPLIT, CHUNK - SPLIT)]],
                         rows_v.at[b, pl.ds(SPLIT, CHUNK - SPLIT)], gsem2.at[b])

    def group(g, p):
        q = (p + 1) % 2

        @pl.when(g >= 2)
        def _():
            pltpu.make_async_copy(
                dst_hbm.at[w, g], dstb.at[p], isem.at[p, 1]).wait()

        for b in range(NBUF):
            i = g * NBUF + b
            pltpu.make_async_copy(
                x_hbm.at[srcb.at[p, b, pl.ds(0, SPLIT)]],
                rows_v.at[b, pl.ds(0, SPLIT)], gsem.at[b]).wait()
            pltpu.make_async_copy(
                x_hbm.at[srcb.at[p, b, pl.ds(SPLIT, CHUNK - SPLIT)]],
                rows_v.at[b, pl.ds(SPLIT, CHUNK - SPLIT)], gsem2.at[b]).wait()
            pltpu.async_copy(rows_v.at[b], acc.at[dstb.at[p, b]],
                             ssem.at[b], add=True)

            if b == 2:
                @pl.when(jnp.logical_and(g >= 1, g + 1 < NGROUPS))
                def _():
                    pltpu.make_async_copy(
                        src_hbm.at[w, g + 1], srcb.at[q], isem.at[q, 0]).wait()

            bj = (b + DEFER) % NBUF
            j = i + DEFER
            pj, pw = (p, q) if b < NBUF - DEFER else (q, p)

            @pl.when(jnp.logical_and(j >= NBUF, j < N_CHUNKS))
            def _():
                pltpu.make_async_copy(
                    rows_v.at[bj], acc.at[dstb.at[pw, bj]], ssem.at[bj]).wait()
                pltpu.async_copy(
                    x_hbm.at[srcb.at[pj, bj, pl.ds(0, SPLIT)]],
                    rows_v.at[bj, pl.ds(0, SPLIT)], gsem.at[bj])
                pltpu.async_copy(
                    x_hbm.at[srcb.at[pj, bj, pl.ds(SPLIT, CHUNK - SPLIT)]],
                    rows_v.at[bj, pl.ds(SPLIT, CHUNK - SPLIT)], gsem2.at[bj])

            if b == 1:
                @pl.when(jnp.logical_and(g + 1 >= 2, g + 1 < NGROUPS))
                def _():
                    pltpu.async_copy(
                        dst_hbm.at[w, g + 1], dstb.at[q], isem.at[q, 1])
            if b == 4:
                @pl.when(jnp.logical_and(g + 2 >= 2, g + 2 < NGROUPS))
                def _():
                    pltpu.async_copy(
                        src_hbm.at[w, g + 2], srcb.at[p], isem.at[p, 0])

    def body(t, carry):
        group(2 * t, 0)
        group(2 * t + 1, 1)
        return carry

    lax.fori_loop(0, NGROUPS // 2, body, 0)
    for b in range(NBUF):
        pltpu.make_async_copy(
            rows_v.at[b], acc.at[dstb.at[(NGROUPS - 1) % 2, b]],
            ssem.at[b]).wait()
    plsc.subcore_barrier()
    pltpu.sync_copy(acc.at[pl.ds(r0, ROWS_PER_TILE)],
                    out_hbm.at[c, pl.ds(r0, ROWS_PER_TILE)])


_segsum_call = pl.kernel(
    _segsum_body,
    out_type=jax.ShapeDtypeStruct((NC, N_PAD, NF), jnp.float32),
    mesh=plsc.VectorSubcoreMesh(core_axis_name="c", subcore_axis_name="s"),
    scratch_types=[
        pltpu.VMEM_SHARED((N_PAD, NF), jnp.float32),
        pltpu.VMEM((2, NBUF, CHUNK), jnp.int32),
        pltpu.VMEM((2, NBUF, CHUNK), jnp.int32),
        pltpu.VMEM((NBUF, CHUNK, NF), jnp.float32),
        pltpu.SemaphoreType.DMA((NBUF,)),
        pltpu.SemaphoreType.DMA((NBUF,)),
        pltpu.SemaphoreType.DMA((NBUF,)),
        pltpu.SemaphoreType.DMA((2, 2)),
    ],
)


def _elu(a):
    return jnp.where(a > 0, a, jnp.exp(a) - 1.0)


def _mlp_body(p_ref, w1_ref, w2_ref, out_ref):
    a = p_ref[0] + p_ref[1]
    h1 = _elu(jnp.dot(a, w1_ref[...], preferred_element_type=jnp.float32))
    out_ref[...] = jnp.dot(h1, w2_ref[...], preferred_element_type=jnp.float32)


def _head_body(p_ref, fw_ref, fb_ref, out_ref):
    z = _elu(p_ref[0] + p_ref[1])
    logits = jnp.dot(z, fw_ref[...], preferred_element_type=jnp.float32)
    logits = logits + fb_ref[...]
    m = jnp.max(logits, axis=1, keepdims=True)
    lse = jnp.log(jnp.sum(jnp.exp(logits - m), axis=1, keepdims=True)) + m
    out_ref[...] = logits - lse


_BLK = 5056
_GRID = N_PAD // _BLK


def _mlp(p, W1, W2):
    return pl.pallas_call(
        _mlp_body,
        grid=(_GRID,),
        in_specs=[
            pl.BlockSpec((2, _BLK, NF), lambda i: (0, i, 0)),
            pl.BlockSpec((NF, NF), lambda i: (0, 0)),
            pl.BlockSpec((NF, NF), lambda i: (0, 0)),
        ],
        out_specs=pl.BlockSpec((_BLK, NF), lambda i: (i, 0)),
        out_shape=jax.ShapeDtypeStruct((N_PAD, NF), jnp.float32),
    )(p, W1, W2)


def _head(p, fc_w, fc_b):
    return pl.pallas_call(
        _head_body,
        grid=(_GRID,),
        in_specs=[
            pl.BlockSpec((2, _BLK, NF), lambda i: (0, i, 0)),
            pl.BlockSpec((NF, NCLASS), lambda i: (0, 0)),
            pl.BlockSpec((1, NCLASS), lambda i: (0, 0)),
        ],
        out_specs=pl.BlockSpec((_BLK, NCLASS), lambda i: (i, 0)),
        out_shape=jax.ShapeDtypeStruct((N, NCLASS), jnp.float32),
    )(p, fc_w, fc_b)


def kernel(x, edge_index, W1, W2, fc_w, fc_b):
    e = edge_index.astype(jnp.int32).reshape(2, NW, NGROUPS, NBUF, CHUNK)
    zeros = jnp.zeros((N_PAD, NF), jnp.float32)

    p = _segsum_call(x, e, zeros)
    h2 = _mlp(p, W1, W2)
    q = _segsum_call(h2, e, zeros)
    return _head(q, fc_w, fc_b.reshape(1, NCLASS))
